# baseline (speedup 1.0000x reference)
import functools

import jax
import jax.numpy as jnp
from jax import lax
from jax.experimental import pallas as pl
from jax.experimental.pallas import tpu as pltpu
from jax.experimental.pallas import tpu_sc as plsc

N = 10000
NP = 10240
K = 20
G = 10
M = N * K
MF = float(M)
T = 512
NT = NP // T
TE = 2560
NTE = NP // TE
TN = 4000
NN = M // TN
NPT = TN // K
BN_ = M + 192
_INF = float("inf")
_IMAX = 2**31 - 1
_EPS = 1e-5


def _knn_body(rmin_ref, rmax_ref, x_ref, xT_ref, bcol_ref, brow_ref,
              out_ref, topd, topi):
    r = pl.program_id(0)
    c = pl.program_id(1)

    @pl.when(c == 0)
    def _init():
        topd[...] = jnp.full((T, 32), _INF, jnp.float32)
        topi[...] = jnp.full((T, 32), _IMAX, jnp.int32)

    active = jnp.logical_and(rmin_ref[c] <= rmax_ref[r],
                             rmax_ref[c] >= rmin_ref[r])

    @pl.when(active)
    def _merge():
        xr = x_ref[pl.ds(r * T, T), :]
        xc = xT_ref[:, pl.ds(c * T, T)]
        xxr = jnp.sum(xr * xr, axis=1, keepdims=True)
        xxc = jnp.sum(xc * xc, axis=0, keepdims=True)
        d = xxr - 2.0 * jnp.dot(xr, xc, preferred_element_type=jnp.float32) + xxc
        same = bcol_ref[pl.ds(r * T, T), :] == brow_ref[:, pl.ds(c * T, T)]
        d = jnp.where(same, d, _INF)
        ids = c * T + lax.broadcasted_iota(jnp.int32, (T, T), 1)
        cd = jnp.concatenate([topd[...], d], axis=1)
        ci = jnp.concatenate([topi[...], ids], axis=1)
        nd = []
        ni = []
        for _ in range(K):
            m = jnp.min(cd, axis=1, keepdims=True)
            sel = jnp.min(jnp.where(cd == m, ci, _IMAX), axis=1, keepdims=True)
            nd.append(m)
            ni.append(sel)
            cd = jnp.where(ci == sel, _INF, cd)
        pad_d = jnp.full((T, 32 - K), _INF, jnp.float32)
        pad_i = jnp.full((T, 32 - K), _IMAX, jnp.int32)
        topd[...] = jnp.concatenate(nd + [pad_d], axis=1)
        topi[...] = jnp.concatenate(ni + [pad_i], axis=1)

    out_ref[...] = topi[...]


def _knn(xp, xpT, bcol, brow, rmin, rmax):
    return pl.pallas_call(
        _knn_body,
        grid=(NT, NT),
        in_specs=[
            pl.BlockSpec(memory_space=pltpu.SMEM),
            pl.BlockSpec(memory_space=pltpu.SMEM),
            pl.BlockSpec((NP, 128), lambda r, c: (0, 0)),
            pl.BlockSpec((128, NP), lambda r, c: (0, 0)),
            pl.BlockSpec((NP, 1), lambda r, c: (0, 0)),
            pl.BlockSpec((1, NP), lambda r, c: (0, 0)),
        ],
        out_specs=pl.BlockSpec((T, 32), lambda r, c: (r, 0)),
        out_shape=jax.ShapeDtypeStruct((NP, 32), jnp.int32),
        scratch_shapes=[
            pltpu.VMEM((T, 32), jnp.float32),
            pltpu.VMEM((T, 32), jnp.int32),
        ],
    )(rmin, rmax, xp, xpT, bcol, brow)


def _sc_gather(table, idx, chunk):
    b = idx.shape[0]
    d = table.shape[1]
    nw = 32
    b_per_w = b // nw
    nchunks = b_per_w // chunk
    mesh = plsc.VectorSubcoreMesh(core_axis_name="c", subcore_axis_name="s")

    @functools.partial(
        pl.kernel,
        mesh=mesh,
        out_type=jax.ShapeDtypeStruct((b, d), jnp.float32),
        scratch_types=[
            pltpu.VMEM((chunk,), jnp.int32),
            pltpu.VMEM((chunk, d), jnp.float32),
            pltpu.SemaphoreType.DMA,
        ],
    )
    def gather_kernel(table_hbm, idx_hbm, out_hbm, idx_v, rows_v, sem):
        wid = lax.axis_index("s") * 2 + lax.axis_index("c")
        base = wid * b_per_w
        for ci in range(nchunks):
            off = base + ci * chunk
            pltpu.sync_copy(idx_hbm.at[pl.ds(off, chunk)], idx_v)
            pltpu.async_copy(table_hbm.at[idx_v], rows_v, sem).wait()
            pltpu.sync_copy(rows_v, out_hbm.at[pl.ds(off, chunk)])

    return gather_kernel(table, idx)


def _edge1(xi, xj, w1_ref, pv_ref):
    col = lax.broadcasted_iota(jnp.int32, (1, 128), 1)
    e = jnp.where(col < 8, xi, 0.0) + jnp.where(
        jnp.logical_and(col >= 8, col < 16), xj - xi, 0.0)
    return jnp.maximum(
        jnp.dot(e, w1_ref[...], preferred_element_type=jnp.float32)
        + pv_ref[0:1, :], 0.0)


def _mlp1_stats_body(x_ref, xjn_ref, ee_ref, w1_ref, w2_ref, w3_ref, pv_ref,
                     so_ref, acc):
    p = pl.program_id(0)
    t = pl.program_id(1)

    @pl.when(jnp.logical_and(p == 0, t == 0))
    def _init():
        acc[...] = jnp.zeros((8, 64), jnp.float32)

    xi = jnp.dot(ee_ref[...], x_ref[pl.ds(t * NPT, NPT), :],
                 preferred_element_type=jnp.float32,
                 precision=lax.Precision.HIGHEST)
    a1 = _edge1(xi, xjn_ref[...], w1_ref, pv_ref)

    def tsum(v):
        return jnp.sum(v, axis=0, keepdims=True)

    def mean_of(j):
        return acc[j:j + 1, :] / MF

    def bn(a, j, grow, berow):
        return (pv_ref[grow:grow + 1, :] * (a - mean_of(j))
                * lax.rsqrt(acc[j + 1:j + 2, :] / MF + _EPS)
                + pv_ref[berow:berow + 1, :])

    @pl.when(p == 0)
    def _s1():
        acc[0:1, :] += tsum(a1)

    @pl.when(p == 1)
    def _v1():
        d = a1 - mean_of(0)
        acc[1:2, :] += tsum(d * d)

    @pl.when(p >= 2)
    def _l2():
        a2 = jnp.maximum(
            jnp.dot(bn(a1, 0, 1, 2), w2_ref[...],
                    preferred_element_type=jnp.float32) + pv_ref[3:4, :], 0.0)

        @pl.when(p == 2)
        def _s2():
            acc[2:3, :] += tsum(a2)

        @pl.when(p == 3)
        def _v2():
            d = a2 - mean_of(2)
            acc[3:4, :] += tsum(d * d)

        @pl.when(p >= 4)
        def _l3():
            a3 = jnp.maximum(
                jnp.dot(bn(a2, 2, 4, 5), w3_ref[...],
                        preferred_element_type=jnp.float32)
                + pv_ref[6:7, :], 0.0)

            @pl.when(p == 4)
            def _s3():
                acc[4:5, :] += tsum(a3)

            @pl.when(p == 5)
            def _v3():
                d = a3 - mean_of(4)
                acc[5:6, :] += tsum(d * d)

    so_ref[...] = acc[...]


def _mlp1_stats(xp2, xjn, ee, w1, w2, w3, pv):
    return pl.pallas_call(
        _mlp1_stats_body,
        grid=(6, NN),
        in_specs=[
            pl.BlockSpec((NP, 128), lambda p, t: (0, 0)),
            pl.BlockSpec((TN, 128), lambda p, t: (t, 0)),
            pl.BlockSpec((TN, NPT), lambda p, t: (0, 0)),
            pl.BlockSpec((128, 64), lambda p, t: (0, 0)),
            pl.BlockSpec((64, 64), lambda p, t: (0, 0)),
            pl.BlockSpec((64, 64), lambda p, t: (0, 0)),
            pl.BlockSpec((16, 64), lambda p, t: (0, 0)),
        ],
        out_specs=pl.BlockSpec((8, 64), lambda p, t: (0, 0)),
        out_shape=jax.ShapeDtypeStruct((8, 64), jnp.float32),
        scratch_shapes=[pltpu.VMEM((8, 64), jnp.float32)],
    )(xp2, xjn, ee, w1, w2, w3, pv)


def _mlp1_final_body(x_ref, xj_ref, w1_ref, w2_ref, w3_ref, pv_ref, st_ref,
                     out_ref):
    k = pl.program_id(0)
    t = pl.program_id(1)

    def bn(a, j, grow, berow):
        return (pv_ref[grow:grow + 1, :] * (a - st_ref[j:j + 1, :] / MF)
                * lax.rsqrt(st_ref[j + 1:j + 2, :] / MF + _EPS)
                + pv_ref[berow:berow + 1, :])

    a1 = _edge1(x_ref[pl.ds(t * TE, TE), :], xj_ref[...], w1_ref, pv_ref)
    a2 = jnp.maximum(
        jnp.dot(bn(a1, 0, 1, 2), w2_ref[...],
                preferred_element_type=jnp.float32) + pv_ref[3:4, :], 0.0)
    a3 = jnp.maximum(
        jnp.dot(bn(a2, 2, 4, 5), w3_ref[...],
                preferred_element_type=jnp.float32) + pv_ref[6:7, :], 0.0)
    h3 = bn(a3, 4, 7, 8)
    h3p = jnp.concatenate([h3, h3], axis=1)

    @pl.when(k == 0)
    def _first():
        out_ref[pl.ds(t * TE, TE), :] = h3p

    @pl.when(k > 0)
    def _acc():
        cur = out_ref[pl.ds(t * TE, TE), :]
        out_ref[pl.ds(t * TE, TE), :] = jnp.maximum(cur, h3p)


def _mlp1_final(xp2, xj, w1, w2, w3, pv, st):
    return pl.pallas_call(
        _mlp1_final_body,
        grid=(K, NTE),
        in_specs=[
            pl.BlockSpec((NP, 128), lambda k, t: (0, 0)),
            pl.BlockSpec((TE, 128), lambda k, t: (k * NTE + t, 0)),
            pl.BlockSpec((128, 64), lambda k, t: (0, 0)),
            pl.BlockSpec((64, 64), lambda k, t: (0, 0)),
            pl.BlockSpec((64, 64), lambda k, t: (0, 0)),
            pl.BlockSpec((16, 64), lambda k, t: (0, 0)),
            pl.BlockSpec((8, 64), lambda k, t: (0, 0)),
        ],
        out_specs=pl.BlockSpec((NP, 128), lambda k, t: (0, 0)),
        out_shape=jax.ShapeDtypeStruct((NP, 128), jnp.float32),
    )(xp2, xj, w1, w2, w3, pv, st)


def _edge2(hi, hj, w4_ref, pv_ref):
    col = lax.broadcasted_iota(jnp.int32, (1, 128), 1)
    e = jnp.where(col < 64, hi, 0.0) + jnp.where(col >= 64, hj - hi, 0.0)
    return jnp.maximum(
        jnp.dot(e, w4_ref[...], preferred_element_type=jnp.float32)
        + pv_ref[0:1, :], 0.0)


def _mlp2_stats_body(h_ref, hjn_ref, ee_ref, w4_ref, pv_ref, so_ref, acc):
    p = pl.program_id(0)
    t = pl.program_id(1)

    @pl.when(jnp.logical_and(p == 0, t == 0))
    def _init():
        acc[...] = jnp.zeros((8, 128), jnp.float32)

    hi = jnp.dot(ee_ref[...], h_ref[pl.ds(t * NPT, NPT), :],
                 preferred_element_type=jnp.float32,
                 precision=lax.Precision.HIGHEST)
    a4 = _edge2(hi, hjn_ref[...], w4_ref, pv_ref)

    @pl.when(p == 0)
    def _s4():
        acc[0:1, :] += jnp.sum(a4, axis=0, keepdims=True)

    @pl.when(p == 1)
    def _v4():
        d = a4 - acc[0:1, :] / MF
        acc[1:2, :] += jnp.sum(d * d, axis=0, keepdims=True)

    so_ref[...] = acc[...]


def _mlp2_stats(hp, hjn, ee, w4, pv):
    return pl.pallas_call(
        _mlp2_stats_body,
        grid=(2, NN),
        in_specs=[
            pl.BlockSpec((NP, 128), lambda p, t: (0, 0)),
            pl.BlockSpec((TN, 128), lambda p, t: (t, 0)),
            pl.BlockSpec((TN, NPT), lambda p, t: (0, 0)),
            pl.BlockSpec((128, 128), lambda p, t: (0, 0)),
            pl.BlockSpec((8, 128), lambda p, t: (0, 0)),
        ],
        out_specs=pl.BlockSpec((8, 128), lambda p, t: (0, 0)),
        out_shape=jax.ShapeDtypeStruct((8, 128), jnp.float32),
        scratch_shapes=[pltpu.VMEM((8, 128), jnp.float32)],
    )(hp, hjn, ee, w4, pv)


def _mlp2_final_body(h_ref, hj_ref, w4_ref, pv_ref, wl_ref, bcol_ref, st_ref,
                     out_ref, h2acc, pooled):
    k = pl.program_id(0)
    t = pl.program_id(1)

    a4 = _edge2(h_ref[pl.ds(t * TE, TE), :], hj_ref[...], w4_ref, pv_ref)
    m = st_ref[0:1, :] / MF
    v = st_ref[1:2, :] / MF
    h = pv_ref[1:2, :] * (a4 - m) * lax.rsqrt(v + _EPS) + pv_ref[2:3, :]
    rows = t * TE + lax.broadcasted_iota(jnp.int32, (TE, 1), 0)
    mask = rows < N

    @pl.when(k == 0)
    def _first():
        h2acc[pl.ds(t * TE, TE), :] = h

    @pl.when(k > 0)
    def _acc():
        cur = h2acc[pl.ds(t * TE, TE), :]
        h2acc[pl.ds(t * TE, TE), :] = jnp.maximum(cur, h)

    @pl.when(k == K - 1)
    def _pool():
        @pl.when(t == 0)
        def _pinit():
            pooled[...] = jnp.full((16, 128), -_INF, jnp.float32)

        hcur = h2acc[pl.ds(t * TE, TE), :]
        bt = bcol_ref[pl.ds(t * TE, TE), :]
        for g in range(G):
            sel = jnp.logical_and(bt == g, mask)
            mg = jnp.max(jnp.where(sel, hcur, -_INF), axis=0, keepdims=True)
            pooled[g:g + 1, :] = jnp.maximum(pooled[g:g + 1, :], mg)

        @pl.when(t == NTE - 1)
        def _head():
            out_ref[...] = jnp.dot(
                pooled[...], wl_ref[...],
                preferred_element_type=jnp.float32) + pv_ref[3:4, :]


def _mlp2_final(hp, hj, w4, pv, wl, bcol, st):
    return pl.pallas_call(
        _mlp2_final_body,
        grid=(K, NTE),
        in_specs=[
            pl.BlockSpec((NP, 128), lambda k, t: (0, 0)),
            pl.BlockSpec((TE, 128), lambda k, t: (k * NTE + t, 0)),
            pl.BlockSpec((128, 128), lambda k, t: (0, 0)),
            pl.BlockSpec((8, 128), lambda k, t: (0, 0)),
            pl.BlockSpec((128, 128), lambda k, t: (0, 0)),
            pl.BlockSpec((NP, 1), lambda k, t: (0, 0)),
            pl.BlockSpec((8, 128), lambda k, t: (0, 0)),
        ],
        out_specs=pl.BlockSpec((16, 128), lambda k, t: (0, 0)),
        out_shape=jax.ShapeDtypeStruct((16, 128), jnp.float32),
        scratch_shapes=[
            pltpu.VMEM((NP, 128), jnp.float32),
            pltpu.VMEM((16, 128), jnp.float32),
        ],
    )(hp, hj, w4, pv, wl, bcol, st)


def _knn_idx(feat, featT, bcol, brow, rmin, rmax):
    idx = _knn(feat, featT, bcol, brow, rmin, rmax)[:, :K]
    idx = jnp.where(jnp.arange(NP)[:, None] < N, idx, 0)
    return jnp.clip(idx, 0, NP - 1)


def kernel(x, batch, W1, b1, g1, be1, W2, b2, g2, be2, W3, b3, g3, be3,
           W4, b4, g4, be4, Wl, bl):
    batch = batch.astype(jnp.int32)
    batchp = jnp.concatenate([batch, jnp.full((NP - N,), G, jnp.int32)])
    bcol = batchp.reshape(NP, 1)
    brow = batchp.reshape(1, NP)
    rmin = batchp[0::T]
    rmax = batchp[T - 1::T]

    xpz = jnp.zeros((NP, 128), jnp.float32).at[:N, :8].set(x)
    xp2 = xpz.at[:N, 8:16].set(x)

    w1 = jnp.zeros((128, 64), jnp.float32).at[:16, :].set(W1)
    pv1 = jnp.zeros((16, 64), jnp.float32)
    pv1 = pv1.at[0].set(b1).at[1].set(g1).at[2].set(be1)
    pv1 = pv1.at[3].set(b2).at[4].set(g2).at[5].set(be2)
    pv1 = pv1.at[6].set(b3).at[7].set(g3).at[8].set(be3)

    ee = jnp.zeros((TN, NPT), jnp.float32).at[
        jnp.arange(TN), jnp.arange(TN) // K].set(1.0)

    idx1 = _knn_idx(xpz, xpz.T, bcol, brow, rmin, rmax)
    idx1n = jnp.concatenate(
        [idx1[:N].reshape(-1), jnp.zeros((BN_ - M,), jnp.int32)])
    xjn = _sc_gather(xp2, idx1n, 272)
    xj = _sc_gather(xp2, idx1.T.reshape(-1), 800)
    st1 = _mlp1_stats(xp2, xjn, ee, w1, W2, W3, pv1)
    hp = _mlp1_final(xp2, xj, w1, W2, W3, pv1, st1)

    pv2 = jnp.zeros((8, 128), jnp.float32)
    pv2 = pv2.at[0].set(b4).at[1].set(g4).at[2].set(be4)
    pv2 = pv2.at[3, :2].set(bl)
    wlp = jnp.zeros((128, 128), jnp.float32).at[:, :2].set(Wl)

    h1z = hp.at[:, 64:].set(0.0)
    idx2 = _knn_idx(h1z, h1z.T, bcol, brow, rmin, rmax)
    idx2n = jnp.concatenate(
        [idx2[:N].reshape(-1), jnp.zeros((BN_ - M,), jnp.int32)])
    hjn = _sc_gather(hp, idx2n, 272)
    hj = _sc_gather(hp, idx2.T.reshape(-1), 800)
    st2 = _mlp2_stats(hp, hjn, ee, W4, pv2)
    out = _mlp2_final(hp, hj, W4, pv2, wlp, bcol, st2)
    return out[:G, :2]


# speedup vs baseline: 1.0002x; 1.0002x over previous
import functools

import jax
import jax.numpy as jnp
from jax import lax
from jax.experimental import pallas as pl
from jax.experimental.pallas import tpu as pltpu
from jax.experimental.pallas import tpu_sc as plsc

N = 10000
NP = 10240
K = 20
G = 10
M = N * K
MF = float(M)
T = 512
NT = NP // T
TE = 2560
NTE = NP // TE
TN = 4000
NN = M // TN
NPT = TN // K
BN_ = M + 192
_INF = float("inf")
_IMAX = 2**31 - 1
_EPS = 1e-5


def _knn_body(rmin_ref, rmax_ref, x_ref, xT_ref, bcol_ref, brow_ref,
              out_ref, topd, topi):
    r = pl.program_id(0)
    c = pl.program_id(1)

    @pl.when(c == 0)
    def _init():
        topd[...] = jnp.full((T, 32), _INF, jnp.float32)
        topi[...] = jnp.full((T, 32), _IMAX, jnp.int32)

    active = jnp.logical_and(rmin_ref[c] <= rmax_ref[r],
                             rmax_ref[c] >= rmin_ref[r])

    @pl.when(active)
    def _merge():
        xr = x_ref[pl.ds(r * T, T), :]
        xc = xT_ref[:, pl.ds(c * T, T)]
        xxr = jnp.sum(xr * xr, axis=1, keepdims=True)
        xxc = jnp.sum(xc * xc, axis=0, keepdims=True)
        d = xxr - 2.0 * jnp.dot(xr, xc, preferred_element_type=jnp.float32) + xxc
        same = bcol_ref[pl.ds(r * T, T), :] == brow_ref[:, pl.ds(c * T, T)]
        d = jnp.where(same, d, _INF)
        ids = c * T + lax.broadcasted_iota(jnp.int32, (T, T), 1)
        cd = jnp.concatenate([topd[...], d], axis=1)
        ci = jnp.concatenate([topi[...], ids], axis=1)
        nd = []
        ni = []
        for _ in range(K):
            m = jnp.min(cd, axis=1, keepdims=True)
            sel = jnp.min(jnp.where(cd == m, ci, _IMAX), axis=1, keepdims=True)
            nd.append(m)
            ni.append(sel)
            cd = jnp.where(ci == sel, _INF, cd)
        pad_d = jnp.full((T, 32 - K), _INF, jnp.float32)
        pad_i = jnp.full((T, 32 - K), _IMAX, jnp.int32)
        topd[...] = jnp.concatenate(nd + [pad_d], axis=1)
        topi[...] = jnp.concatenate(ni + [pad_i], axis=1)

    out_ref[...] = topi[...]


def _knn(xp, xpT, bcol, brow, rmin, rmax):
    return pl.pallas_call(
        _knn_body,
        grid=(NT, NT),
        in_specs=[
            pl.BlockSpec(memory_space=pltpu.SMEM),
            pl.BlockSpec(memory_space=pltpu.SMEM),
            pl.BlockSpec((NP, 128), lambda r, c: (0, 0)),
            pl.BlockSpec((128, NP), lambda r, c: (0, 0)),
            pl.BlockSpec((NP, 1), lambda r, c: (0, 0)),
            pl.BlockSpec((1, NP), lambda r, c: (0, 0)),
        ],
        out_specs=pl.BlockSpec((T, 32), lambda r, c: (r, 0)),
        out_shape=jax.ShapeDtypeStruct((NP, 32), jnp.int32),
        scratch_shapes=[
            pltpu.VMEM((T, 32), jnp.float32),
            pltpu.VMEM((T, 32), jnp.int32),
        ],
    )(rmin, rmax, xp, xpT, bcol, brow)


def _sc_gather(table, idx, chunk):
    b = idx.shape[0]
    d = table.shape[1]
    nw = 32
    b_per_w = b // nw
    nchunks = b_per_w // chunk
    mesh = plsc.VectorSubcoreMesh(core_axis_name="c", subcore_axis_name="s")

    @functools.partial(
        pl.kernel,
        mesh=mesh,
        out_type=jax.ShapeDtypeStruct((b, d), jnp.float32),
        scratch_types=[
            pltpu.VMEM((chunk,), jnp.int32),
            pltpu.VMEM((chunk, d), jnp.float32),
            pltpu.SemaphoreType.DMA,
        ],
    )
    def gather_kernel(table_hbm, idx_hbm, out_hbm, idx_v, rows_v, sem):
        wid = lax.axis_index("s") * 2 + lax.axis_index("c")
        base = wid * b_per_w
        for ci in range(nchunks):
            off = base + ci * chunk
            pltpu.sync_copy(idx_hbm.at[pl.ds(off, chunk)], idx_v)
            pltpu.async_copy(table_hbm.at[idx_v], rows_v, sem).wait()
            pltpu.sync_copy(rows_v, out_hbm.at[pl.ds(off, chunk)])

    return gather_kernel(table, idx)


def _edge1(xi, xj, w1_ref, pv_ref):
    col = lax.broadcasted_iota(jnp.int32, (1, 128), 1)
    e = jnp.where(col < 8, xi, 0.0) + jnp.where(
        jnp.logical_and(col >= 8, col < 16), xj - xi, 0.0)
    return jnp.maximum(
        jnp.dot(e, w1_ref[...], preferred_element_type=jnp.float32)
        + pv_ref[0:1, :], 0.0)


def _mlp1_stats_body(x_ref, xjn_ref, ee_ref, w1_ref, w2_ref, w3_ref, pv_ref,
                     so_ref, acc):
    p = pl.program_id(0)
    t = pl.program_id(1)

    @pl.when(jnp.logical_and(p == 0, t == 0))
    def _init():
        acc[...] = jnp.zeros((8, 64), jnp.float32)

    xi = jnp.dot(ee_ref[...], x_ref[pl.ds(t * NPT, NPT), :],
                 preferred_element_type=jnp.float32,
                 precision=lax.Precision.HIGHEST)
    a1 = _edge1(xi, xjn_ref[...], w1_ref, pv_ref)

    def tsum(v):
        return jnp.sum(v, axis=0, keepdims=True)

    def mean_of(j):
        return acc[j:j + 1, :] / MF

    def bn(a, j, grow, berow):
        return (pv_ref[grow:grow + 1, :] * (a - mean_of(j))
                * lax.rsqrt(acc[j + 1:j + 2, :] / MF + _EPS)
                + pv_ref[berow:berow + 1, :])

    @pl.when(p == 0)
    def _s1():
        acc[0:1, :] += tsum(a1)

    @pl.when(p == 1)
    def _v1():
        d = a1 - mean_of(0)
        acc[1:2, :] += tsum(d * d)

    @pl.when(p >= 2)
    def _l2():
        a2 = jnp.maximum(
            jnp.dot(bn(a1, 0, 1, 2), w2_ref[...],
                    preferred_element_type=jnp.float32) + pv_ref[3:4, :], 0.0)

        @pl.when(p == 2)
        def _s2():
            acc[2:3, :] += tsum(a2)

        @pl.when(p == 3)
        def _v2():
            d = a2 - mean_of(2)
            acc[3:4, :] += tsum(d * d)

        @pl.when(p >= 4)
        def _l3():
            a3 = jnp.maximum(
                jnp.dot(bn(a2, 2, 4, 5), w3_ref[...],
                        preferred_element_type=jnp.float32)
                + pv_ref[6:7, :], 0.0)

            @pl.when(p == 4)
            def _s3():
                acc[4:5, :] += tsum(a3)

            @pl.when(p == 5)
            def _v3():
                d = a3 - mean_of(4)
                acc[5:6, :] += tsum(d * d)

    so_ref[...] = acc[...]


def _mlp1_stats(xp2, xjn, ee, w1, w2, w3, pv):
    return pl.pallas_call(
        _mlp1_stats_body,
        grid=(6, NN),
        in_specs=[
            pl.BlockSpec((NP, 128), lambda p, t: (0, 0)),
            pl.BlockSpec((TN, 128), lambda p, t: (t, 0)),
            pl.BlockSpec((TN, NPT), lambda p, t: (0, 0)),
            pl.BlockSpec((128, 64), lambda p, t: (0, 0)),
            pl.BlockSpec((64, 64), lambda p, t: (0, 0)),
            pl.BlockSpec((64, 64), lambda p, t: (0, 0)),
            pl.BlockSpec((16, 64), lambda p, t: (0, 0)),
        ],
        out_specs=pl.BlockSpec((8, 64), lambda p, t: (0, 0)),
        out_shape=jax.ShapeDtypeStruct((8, 64), jnp.float32),
        scratch_shapes=[pltpu.VMEM((8, 64), jnp.float32)],
    )(xp2, xjn, ee, w1, w2, w3, pv)


def _mlp1_final_body(x_ref, xj_ref, w1_ref, w2_ref, w3_ref, pv_ref, st_ref,
                     out_ref):
    k = pl.program_id(0)
    t = pl.program_id(1)

    def bn(a, j, grow, berow):
        return (pv_ref[grow:grow + 1, :] * (a - st_ref[j:j + 1, :] / MF)
                * lax.rsqrt(st_ref[j + 1:j + 2, :] / MF + _EPS)
                + pv_ref[berow:berow + 1, :])

    a1 = _edge1(x_ref[pl.ds(t * TE, TE), :], xj_ref[...], w1_ref, pv_ref)
    a2 = jnp.maximum(
        jnp.dot(bn(a1, 0, 1, 2), w2_ref[...],
                preferred_element_type=jnp.float32) + pv_ref[3:4, :], 0.0)
    a3 = jnp.maximum(
        jnp.dot(bn(a2, 2, 4, 5), w3_ref[...],
                preferred_element_type=jnp.float32) + pv_ref[6:7, :], 0.0)
    h3 = bn(a3, 4, 7, 8)
    h3p = jnp.concatenate([h3, h3], axis=1)

    @pl.when(k == 0)
    def _first():
        out_ref[pl.ds(t * TE, TE), :] = h3p

    @pl.when(k > 0)
    def _acc():
        cur = out_ref[pl.ds(t * TE, TE), :]
        out_ref[pl.ds(t * TE, TE), :] = jnp.maximum(cur, h3p)


def _mlp1_final(xp2, xj, w1, w2, w3, pv, st):
    return pl.pallas_call(
        _mlp1_final_body,
        grid=(K, NTE),
        in_specs=[
            pl.BlockSpec((NP, 128), lambda k, t: (0, 0)),
            pl.BlockSpec((TE, 128), lambda k, t: (k * NTE + t, 0)),
            pl.BlockSpec((128, 64), lambda k, t: (0, 0)),
            pl.BlockSpec((64, 64), lambda k, t: (0, 0)),
            pl.BlockSpec((64, 64), lambda k, t: (0, 0)),
            pl.BlockSpec((16, 64), lambda k, t: (0, 0)),
            pl.BlockSpec((8, 64), lambda k, t: (0, 0)),
        ],
        out_specs=pl.BlockSpec((NP, 128), lambda k, t: (0, 0)),
        out_shape=jax.ShapeDtypeStruct((NP, 128), jnp.float32),
    )(xp2, xj, w1, w2, w3, pv, st)


def _edge2(hi, hj, w4_ref, pv_ref):
    col = lax.broadcasted_iota(jnp.int32, (1, 128), 1)
    e = jnp.where(col < 64, hi, 0.0) + jnp.where(col >= 64, hj - hi, 0.0)
    return jnp.maximum(
        jnp.dot(e, w4_ref[...], preferred_element_type=jnp.float32)
        + pv_ref[0:1, :], 0.0)


def _mlp2_stats_body(h_ref, hjn_ref, ee_ref, w4_ref, pv_ref, so_ref, acc):
    p = pl.program_id(0)
    t = pl.program_id(1)

    @pl.when(jnp.logical_and(p == 0, t == 0))
    def _init():
        acc[...] = jnp.zeros((8, 128), jnp.float32)

    hi = jnp.dot(ee_ref[...], h_ref[pl.ds(t * NPT, NPT), :],
                 preferred_element_type=jnp.float32,
                 precision=lax.Precision.HIGHEST)
    a4 = _edge2(hi, hjn_ref[...], w4_ref, pv_ref)

    @pl.when(p == 0)
    def _s4():
        acc[0:1, :] += jnp.sum(a4, axis=0, keepdims=True)

    @pl.when(p == 1)
    def _v4():
        d = a4 - acc[0:1, :] / MF
        acc[1:2, :] += jnp.sum(d * d, axis=0, keepdims=True)

    so_ref[...] = acc[...]


def _mlp2_stats(hp, hjn, ee, w4, pv):
    return pl.pallas_call(
        _mlp2_stats_body,
        grid=(2, NN),
        in_specs=[
            pl.BlockSpec((NP, 128), lambda p, t: (0, 0)),
            pl.BlockSpec((TN, 128), lambda p, t: (t, 0)),
            pl.BlockSpec((TN, NPT), lambda p, t: (0, 0)),
            pl.BlockSpec((128, 128), lambda p, t: (0, 0)),
            pl.BlockSpec((8, 128), lambda p, t: (0, 0)),
        ],
        out_specs=pl.BlockSpec((8, 128), lambda p, t: (0, 0)),
        out_shape=jax.ShapeDtypeStruct((8, 128), jnp.float32),
        scratch_shapes=[pltpu.VMEM((8, 128), jnp.float32)],
    )(hp, hjn, ee, w4, pv)


def _mlp2_final_body(h_ref, hj_ref, w4_ref, pv_ref, wl_ref, bcol_ref, st_ref,
                     out_ref, h2acc, pooled):
    k = pl.program_id(0)
    t = pl.program_id(1)

    a4 = _edge2(h_ref[pl.ds(t * TE, TE), :], hj_ref[...], w4_ref, pv_ref)
    m = st_ref[0:1, :] / MF
    v = st_ref[1:2, :] / MF
    h = pv_ref[1:2, :] * (a4 - m) * lax.rsqrt(v + _EPS) + pv_ref[2:3, :]
    rows = t * TE + lax.broadcasted_iota(jnp.int32, (TE, 1), 0)
    mask = rows < N

    @pl.when(k == 0)
    def _first():
        h2acc[pl.ds(t * TE, TE), :] = h

    @pl.when(k > 0)
    def _acc():
        cur = h2acc[pl.ds(t * TE, TE), :]
        h2acc[pl.ds(t * TE, TE), :] = jnp.maximum(cur, h)

    @pl.when(k == K - 1)
    def _pool():
        @pl.when(t == 0)
        def _pinit():
            pooled[...] = jnp.full((16, 128), -_INF, jnp.float32)

        hcur = h2acc[pl.ds(t * TE, TE), :]
        bt = bcol_ref[pl.ds(t * TE, TE), :]
        for g in range(G):
            sel = jnp.logical_and(bt == g, mask)
            mg = jnp.max(jnp.where(sel, hcur, -_INF), axis=0, keepdims=True)
            pooled[g:g + 1, :] = jnp.maximum(pooled[g:g + 1, :], mg)

        @pl.when(t == NTE - 1)
        def _head():
            out_ref[...] = jnp.dot(
                pooled[...], wl_ref[...],
                preferred_element_type=jnp.float32) + pv_ref[3:4, :]


def _mlp2_final(hp, hj, w4, pv, wl, bcol, st):
    return pl.pallas_call(
        _mlp2_final_body,
        grid=(K, NTE),
        in_specs=[
            pl.BlockSpec((NP, 128), lambda k, t: (0, 0)),
            pl.BlockSpec((TE, 128), lambda k, t: (k * NTE + t, 0)),
            pl.BlockSpec((128, 128), lambda k, t: (0, 0)),
            pl.BlockSpec((8, 128), lambda k, t: (0, 0)),
            pl.BlockSpec((128, 128), lambda k, t: (0, 0)),
            pl.BlockSpec((NP, 1), lambda k, t: (0, 0)),
            pl.BlockSpec((8, 128), lambda k, t: (0, 0)),
        ],
        out_specs=pl.BlockSpec((16, 128), lambda k, t: (0, 0)),
        out_shape=jax.ShapeDtypeStruct((16, 128), jnp.float32),
        scratch_shapes=[
            pltpu.VMEM((NP, 128), jnp.float32),
            pltpu.VMEM((16, 128), jnp.float32),
        ],
    )(hp, hj, w4, pv, wl, bcol, st)


def _knn_idx(feat, featT, bcol, brow, rmin, rmax):
    idx = _knn(feat, featT, bcol, brow, rmin, rmax)[:, :K]
    idx = jnp.where(jnp.arange(NP)[:, None] < N, idx, 0)
    return jnp.clip(idx, 0, NP - 1)


def kernel(x, batch, W1, b1, g1, be1, W2, b2, g2, be2, W3, b3, g3, be3,
           W4, b4, g4, be4, Wl, bl):
    batch = batch.astype(jnp.int32)
    batchp = jnp.concatenate([batch, jnp.full((NP - N,), G, jnp.int32)])
    bcol = batchp.reshape(NP, 1)
    brow = batchp.reshape(1, NP)
    rmin = batchp[0::T]
    rmax = batchp[T - 1::T]

    xpz = jnp.zeros((NP, 128), jnp.float32).at[:N, :8].set(x)
    xp2 = xpz.at[:N, 8:16].set(x)

    w1 = jnp.zeros((128, 64), jnp.float32).at[:16, :].set(W1)
    pv1 = jnp.zeros((16, 64), jnp.float32)
    pv1 = pv1.at[0].set(b1).at[1].set(g1).at[2].set(be1)
    pv1 = pv1.at[3].set(b2).at[4].set(g2).at[5].set(be2)
    pv1 = pv1.at[6].set(b3).at[7].set(g3).at[8].set(be3)

    ee = jnp.zeros((TN, NPT), jnp.float32).at[
        jnp.arange(TN), jnp.arange(TN) // K].set(1.0)

    idx1 = _knn_idx(xpz, xpz.T, bcol, brow, rmin, rmax)
    idx1n = jnp.concatenate(
        [idx1[:N].reshape(-1), jnp.zeros((BN_ - M,), jnp.int32)])
    xjn = _sc_gather(xp2, idx1n, 272)
    xj = _sc_gather(xp2, idx1.T.reshape(-1), 320)
    st1 = _mlp1_stats(xp2, xjn, ee, w1, W2, W3, pv1)
    hp = _mlp1_final(xp2, xj, w1, W2, W3, pv1, st1)

    pv2 = jnp.zeros((8, 128), jnp.float32)
    pv2 = pv2.at[0].set(b4).at[1].set(g4).at[2].set(be4)
    pv2 = pv2.at[3, :2].set(bl)
    wlp = jnp.zeros((128, 128), jnp.float32).at[:, :2].set(Wl)

    h1z = hp.at[:, 64:].set(0.0)
    idx2 = _knn_idx(h1z, h1z.T, bcol, brow, rmin, rmax)
    idx2n = jnp.concatenate(
        [idx2[:N].reshape(-1), jnp.zeros((BN_ - M,), jnp.int32)])
    hjn = _sc_gather(hp, idx2n, 272)
    hj = _sc_gather(hp, idx2.T.reshape(-1), 320)
    st2 = _mlp2_stats(hp, hjn, ee, W4, pv2)
    out = _mlp2_final(hp, hj, W4, pv2, wlp, bcol, st2)
    return out[:G, :2]
